# Initial kernel scaffold; baseline (speedup 1.0000x reference)
import jax
import jax.numpy as jnp
from jax.experimental import pallas as pl


def kernel(x, cond, snr, Wg, W1, W2):
    raise NotImplementedError("write your pallas kernel here")


# speedup vs baseline: 2.3360x; 2.3360x over previous
import functools

import jax
import jax.numpy as jnp
from jax.experimental import pallas as pl
from jax.experimental.pallas import tpu as pltpu

_HIDDEN = 1024
_E = 8
_HIGH = 4
_K = 2
_THR = 0.5
_CAP_F = 1.25
_INTER = int(_HIDDEN * 8 // 3)
_IP = 2816
_IB = 704
_NIB = _IP // _IB


def _ffn_kernel(tok_ref, x_ref, w_ref, w1g_ref, w1u_ref, w2t_ref, out_ref,
                xin_ref, eo_ref):
    e = pl.program_id(0)
    ib = pl.program_id(1)
    cap = xin_ref.shape[0]

    @pl.when(jnp.logical_and(e == 0, ib == 0))
    def _():
        out_ref[...] = jnp.zeros_like(out_ref)

    @pl.when(ib == 0)
    def _():
        def gather(i, c):
            t = tok_ref[e * cap + i]
            xin_ref[pl.ds(i, 1), :] = x_ref[pl.ds(t, 1), :]
            return c
        jax.lax.fori_loop(0, cap, gather, 0, unroll=8)

    xin = xin_ref[...]
    w1g = w1g_ref[0, 0]
    w1u = w1u_ref[0, 0]
    g = jax.lax.dot_general(xin, w1g, (((1,), (1,)), ((), ())),
                            preferred_element_type=jnp.float32)
    u = jax.lax.dot_general(xin, w1u, (((1,), (1,)), ((), ())),
                            preferred_element_type=jnp.float32)
    h = g * jax.nn.sigmoid(g) * u
    peo = jax.lax.dot_general(h, w2t_ref[0], (((1,), (0,)), ((), ())),
                              preferred_element_type=jnp.float32)

    @pl.when(ib == 0)
    def _():
        eo_ref[...] = peo

    @pl.when(ib != 0)
    def _():
        eo_ref[...] += peo

    @pl.when(ib == _NIB - 1)
    def _():
        eo_ref[...] *= w_ref[0]

        def scatter(i, c):
            t = tok_ref[e * cap + i]
            out_ref[pl.ds(t, 1), :] = (out_ref[pl.ds(t, 1), :]
                                       + eo_ref[pl.ds(i, 1), :])
            return c
        jax.lax.fori_loop(0, cap, scatter, 0, unroll=8)


def _moe_ffn(tok, w, x_flat, W1p, W2t, cap, interpret=False):
    T, H = x_flat.shape
    grid = (_E, _NIB)
    kernel_fn = _ffn_kernel
    return pl.pallas_call(
        kernel_fn,
        grid_spec=pltpu.PrefetchScalarGridSpec(
            num_scalar_prefetch=1,
            grid=grid,
            in_specs=[
                pl.BlockSpec((T, H), lambda e, ib, tok_ref: (0, 0)),
                pl.BlockSpec((1, cap, 1), lambda e, ib, tok_ref: (e, 0, 0)),
                pl.BlockSpec((1, 1, _IB, H), lambda e, ib, tok_ref: (e, 0, ib, 0)),
                pl.BlockSpec((1, 1, _IB, H), lambda e, ib, tok_ref: (e, 1, ib, 0)),
                pl.BlockSpec((1, _IB, H), lambda e, ib, tok_ref: (e, ib, 0)),
            ],
            out_specs=pl.BlockSpec((T, H), lambda e, ib, tok_ref: (0, 0)),
            scratch_shapes=[
                pltpu.VMEM((cap, H), jnp.float32),
                pltpu.VMEM((cap, H), jnp.float32),
            ],
        ),
        out_shape=jax.ShapeDtypeStruct((T, H), jnp.float32),
        compiler_params=pltpu.CompilerParams(
            dimension_semantics=("arbitrary", "arbitrary"),
        ),
        interpret=interpret,
    )(tok, x_flat, w, W1p, W1p, W2t)


def kernel(x, cond, snr, Wg, W1, W2, interpret=False):
    B, N, C = x.shape
    T = B * N
    x_flat = x.reshape(T, C)
    cond_flat = jnp.repeat(cond, N, axis=0)
    snr_flat = jnp.repeat(snr, N)
    combined = jnp.concatenate([x_flat, cond_flat], axis=-1)
    logits = combined @ Wg.T
    high_noise = (snr_flat < _THR)[:, None]
    neg = jnp.where(high_noise, -jnp.inf, 0.0)
    mask = jnp.concatenate(
        [jnp.zeros((T, _HIGH), dtype=logits.dtype),
         jnp.broadcast_to(neg, (T, _E - _HIGH))],
        axis=1,
    )
    logits = logits + mask
    probs = jax.nn.softmax(logits, axis=-1)
    top_p, top_i = jax.lax.top_k(probs, _K)
    top_p = top_p / jnp.maximum(jnp.sum(jnp.abs(top_p), axis=-1, keepdims=True), 1e-12)
    cap = max(1, int(_CAP_F * T / _E))

    toks = []
    ws = []
    for e in range(_E):
        prio = jnp.where(top_i == e, top_p, -jnp.inf).reshape(-1)
        vals, sel = jax.lax.top_k(prio, cap)
        valid = jnp.isfinite(vals)
        toks.append(jnp.where(valid, sel // _K, 0).astype(jnp.int32))
        ws.append(jnp.where(valid, vals, 0.0))
    tok = jnp.concatenate(toks)
    w = jnp.stack(ws).reshape(_E, cap, 1)

    W1r = W1.reshape(_E, 2, _INTER, C)
    W1p = jnp.pad(W1r, ((0, 0), (0, 0), (0, _IP - _INTER), (0, 0)))
    W2t = jnp.pad(W2.transpose(0, 2, 1), ((0, 0), (0, _IP - _INTER), (0, 0)))

    out = _moe_ffn(tok, w, x_flat, W1p, W2t, cap, interpret=interpret)
    return (out.reshape(B, N, C), jnp.asarray(0.0, dtype=jnp.float32))


# speedup vs baseline: 2.6321x; 1.1267x over previous
import functools

import jax
import jax.numpy as jnp
from jax.experimental import pallas as pl
from jax.experimental.pallas import tpu as pltpu

_HIDDEN = 1024
_E = 8
_HIGH = 4
_K = 2
_THR = 0.5
_CAP_F = 1.25
_INTER = int(_HIDDEN * 8 // 3)
_IB = 512
_NIB = -(-_INTER // _IB)


def _ffn_kernel(tok_ref, x_ref, w_ref, w1g_ref, w1u_ref, w2t_ref, out_ref,
                xin_ref, eo_ref):
    e = pl.program_id(0)
    ib = pl.program_id(1)
    cap = xin_ref.shape[0]

    @pl.when(jnp.logical_and(e == 0, ib == 0))
    def _():
        out_ref[...] = jnp.zeros_like(out_ref)

    @pl.when(ib == 0)
    def _():
        def gather(i, c):
            t = tok_ref[e * cap + i]
            xin_ref[pl.ds(i, 1), :] = x_ref[pl.ds(t, 1), :]
            return c
        jax.lax.fori_loop(0, cap, gather, 0, unroll=8)

    xin = xin_ref[...]
    w1g = w1g_ref[0, 0]
    w1u = w1u_ref[0, 0]
    g = jax.lax.dot_general(xin, w1g, (((1,), (1,)), ((), ())),
                            preferred_element_type=jnp.float32)
    u = jax.lax.dot_general(xin, w1u, (((1,), (1,)), ((), ())),
                            preferred_element_type=jnp.float32)
    h = g * jax.nn.sigmoid(g) * u
    bound = jnp.where(ib == _NIB - 1, _INTER - (_NIB - 1) * _IB, _IB)
    hcol = jax.lax.broadcasted_iota(jnp.int32, h.shape, 1)
    h = jnp.where(hcol < bound, h, 0.0)
    w2 = w2t_ref[0]
    wcol = jax.lax.broadcasted_iota(jnp.int32, w2.shape, 1)
    w2 = jnp.where(wcol < bound, w2, 0.0)
    peo = jax.lax.dot_general(h, w2, (((1,), (1,)), ((), ())),
                              preferred_element_type=jnp.float32)

    @pl.when(ib == 0)
    def _():
        eo_ref[...] = peo

    @pl.when(ib != 0)
    def _():
        eo_ref[...] += peo

    @pl.when(ib == _NIB - 1)
    def _():
        eo_ref[...] *= w_ref[0]

        def scatter(i, c):
            t = tok_ref[e * cap + i]
            out_ref[pl.ds(t, 1), :] = (out_ref[pl.ds(t, 1), :]
                                       + eo_ref[pl.ds(i, 1), :])
            return c
        jax.lax.fori_loop(0, cap, scatter, 0, unroll=8)


def _moe_ffn(tok, w, x_flat, W1p, W2t, cap, interpret=False):
    T, H = x_flat.shape
    grid = (_E, _NIB)
    kernel_fn = _ffn_kernel
    return pl.pallas_call(
        kernel_fn,
        grid_spec=pltpu.PrefetchScalarGridSpec(
            num_scalar_prefetch=1,
            grid=grid,
            in_specs=[
                pl.BlockSpec((T, H), lambda e, ib, tok_ref: (0, 0)),
                pl.BlockSpec((1, cap, 1), lambda e, ib, tok_ref: (e, 0, 0)),
                pl.BlockSpec((1, 1, _IB, H), lambda e, ib, tok_ref: (e, 0, ib, 0)),
                pl.BlockSpec((1, 1, _IB, H), lambda e, ib, tok_ref: (e, 1, ib, 0)),
                pl.BlockSpec((1, H, _IB), lambda e, ib, tok_ref: (e, 0, ib)),
            ],
            out_specs=pl.BlockSpec((T, H), lambda e, ib, tok_ref: (0, 0)),
            scratch_shapes=[
                pltpu.VMEM((cap, H), jnp.float32),
                pltpu.VMEM((cap, H), jnp.float32),
            ],
        ),
        out_shape=jax.ShapeDtypeStruct((T, H), jnp.float32),
        compiler_params=pltpu.CompilerParams(
            dimension_semantics=("arbitrary", "arbitrary"),
        ),
        interpret=interpret,
    )(tok, x_flat, w, W1p, W1p, W2t)


def kernel(x, cond, snr, Wg, W1, W2, interpret=False):
    B, N, C = x.shape
    T = B * N
    x_flat = x.reshape(T, C)
    cond_flat = jnp.repeat(cond, N, axis=0)
    snr_flat = jnp.repeat(snr, N)
    combined = jnp.concatenate([x_flat, cond_flat], axis=-1)
    logits = combined @ Wg.T
    high_noise = (snr_flat < _THR)[:, None]
    neg = jnp.where(high_noise, -jnp.inf, 0.0)
    mask = jnp.concatenate(
        [jnp.zeros((T, _HIGH), dtype=logits.dtype),
         jnp.broadcast_to(neg, (T, _E - _HIGH))],
        axis=1,
    )
    logits = logits + mask
    probs = jax.nn.softmax(logits, axis=-1)
    top_p, top_i = jax.lax.top_k(probs, _K)
    top_p = top_p / jnp.maximum(jnp.sum(jnp.abs(top_p), axis=-1, keepdims=True), 1e-12)
    cap = max(1, int(_CAP_F * T / _E))

    toks = []
    ws = []
    for e in range(_E):
        prio = jnp.where(top_i == e, top_p, -jnp.inf).reshape(-1)
        vals, sel = jax.lax.top_k(prio, cap)
        valid = jnp.isfinite(vals)
        toks.append(jnp.where(valid, sel // _K, 0).astype(jnp.int32))
        ws.append(jnp.where(valid, vals, 0.0))
    tok = jnp.concatenate(toks)
    w = jnp.stack(ws).reshape(_E, cap, 1)

    W1r = W1.reshape(_E, 2, _INTER, C)

    out = _moe_ffn(tok, w, x_flat, W1r, W2, cap, interpret=interpret)
    return (out.reshape(B, N, C), jnp.asarray(0.0, dtype=jnp.float32))
